# Initial kernel scaffold; baseline (speedup 1.0000x reference)
import jax
import jax.numpy as jnp
from jax.experimental import pallas as pl


def kernel(X, hyperedge_index, Theta1, b1, Theta2, b2, Wh, bh):
    raise NotImplementedError("write your pallas kernel here")


# speedup vs baseline: 1.6601x; 1.6601x over previous
import functools

import jax
import jax.numpy as jnp
from jax import lax
from jax.experimental import pallas as pl
from jax.experimental.pallas import tpu as pltpu
from jax.experimental.pallas import tpu_sc as plsc

LB = 128
NC = 2
NS = 16
NW = NC * NS


def _chunks(total, step):
    out = []
    t = 0
    while t < total:
        out.append((t, min(step, total - t)))
        t += step
    return out


@functools.lru_cache(maxsize=None)
def _sc_push(nb, npad):
    rpt = npad // NS
    mesh = plsc.VectorSubcoreMesh(core_axis_name="c", subcore_axis_name="s")

    @functools.partial(
        pl.kernel,
        out_type=jax.ShapeDtypeStruct((NC, npad, LB), jnp.float32),
        mesh=mesh,
        scratch_types=[
            pltpu.VMEM((nb, LB), jnp.int32),
            pltpu.VMEM((nb, LB), jnp.int32),
            pltpu.VMEM((LB, LB), jnp.float32),
            pltpu.VMEM_SHARED((npad, LB), jnp.float32),
            pltpu.SemaphoreType.DMA,
        ],
    )
    def push(y_hbm, sidx_hbm, didx_hbm, zero_hbm, out_hbm,
             sidx, didx, rows, acc, sem):
        c = lax.axis_index("c")
        s = lax.axis_index("s")
        wid = c * NS + s
        pltpu.sync_copy(sidx_hbm.at[wid], sidx)
        pltpu.sync_copy(didx_hbm.at[wid], didx)
        pltpu.sync_copy(zero_hbm, rows)
        base = s * rpt
        for t0, sz in _chunks(rpt, LB):
            pltpu.sync_copy(rows.at[pl.ds(0, sz)], acc.at[pl.ds(base + t0, sz)])
        plsc.subcore_barrier()

        @pl.loop(0, nb)
        def _(j):
            pltpu.async_copy(y_hbm.at[sidx.at[j]], rows, sem).wait()
            pltpu.sync_copy(rows, acc.at[didx.at[j]], add=True)

        plsc.subcore_barrier()
        for t0, sz in _chunks(rpt, LB):
            pltpu.sync_copy(acc.at[pl.ds(base + t0, sz)], rows.at[pl.ds(0, sz)])
            pltpu.sync_copy(rows.at[pl.ds(0, sz)],
                            out_hbm.at[c, pl.ds(base + t0, sz)])

    return push


@functools.lru_cache(maxsize=None)
def _sc_count(nb, npad):
    rpt = npad // NS
    mesh = plsc.VectorSubcoreMesh(core_axis_name="c", subcore_axis_name="s")

    @functools.partial(
        pl.kernel,
        out_type=jax.ShapeDtypeStruct((NC, npad, LB), jnp.float32),
        mesh=mesh,
        scratch_types=[
            pltpu.VMEM((nb, LB), jnp.int32),
            pltpu.VMEM((LB, LB), jnp.float32),
            pltpu.VMEM_SHARED((npad, LB), jnp.float32),
        ],
    )
    def count(idx_hbm, ones_hbm, zero_hbm, out_hbm, idx, buf, acc):
        c = lax.axis_index("c")
        s = lax.axis_index("s")
        wid = c * NS + s
        pltpu.sync_copy(idx_hbm.at[wid], idx)
        pltpu.sync_copy(zero_hbm, buf)
        base = s * rpt
        for t0, sz in _chunks(rpt, LB):
            pltpu.sync_copy(buf.at[pl.ds(0, sz)], acc.at[pl.ds(base + t0, sz)])
        plsc.subcore_barrier()
        pltpu.sync_copy(ones_hbm, buf)

        @pl.loop(0, nb)
        def _(j):
            pltpu.sync_copy(buf, acc.at[idx.at[j]], add=True)

        plsc.subcore_barrier()
        for t0, sz in _chunks(rpt, LB):
            pltpu.sync_copy(acc.at[pl.ds(base + t0, sz)], buf.at[pl.ds(0, sz)])
            pltpu.sync_copy(buf.at[pl.ds(0, sz)],
                            out_hbm.at[c, pl.ds(base + t0, sz)])

    return count


def _scales_y1_body(x_ref, dvp_ref, dep_ref, y1_ref, isdv_ref, ide_ref):
    k = pl.program_id(0)
    dv = dvp_ref[0, :, 0] + dvp_ref[1, :, 0]
    de = dep_ref[0, :, 0] + dep_ref[1, :, 0]
    isdv = jnp.where(dv > 0, lax.rsqrt(jnp.maximum(dv, 1.0)), 0.0)
    ide = jnp.where(de > 0, 1.0 / jnp.maximum(de, 1.0), 0.0)
    isdv_ref[...] = isdv
    ide_ref[...] = ide
    onehot = (lax.broadcasted_iota(jnp.int32, (1, LB), 1) == 0)
    aug = jnp.where(k == 2, onehot.astype(jnp.float32), 0.0)
    y1_ref[...] = ((x_ref[...] + aug) * isdv[:, None])[None]


def _tc_scales_y1(xp, dvp, dep, npad):
    g = npad // LB
    return pl.pallas_call(
        _scales_y1_body,
        grid=(3, g),
        in_specs=[
            pl.BlockSpec((LB, LB), lambda k, i: (i, k)),
            pl.BlockSpec((NC, LB, LB), lambda k, i: (0, i, 0)),
            pl.BlockSpec((NC, LB, LB), lambda k, i: (0, i, 0)),
        ],
        out_specs=(
            pl.BlockSpec((1, LB, LB), lambda k, i: (k, i, 0)),
            pl.BlockSpec((LB,), lambda k, i: (i,)),
            pl.BlockSpec((LB,), lambda k, i: (i,)),
        ),
        out_shape=(
            jax.ShapeDtypeStruct((3, npad, LB), jnp.float32),
            jax.ShapeDtypeStruct((npad,), jnp.float32),
            jax.ShapeDtypeStruct((npad,), jnp.float32),
        ),
    )(xp, dvp, dep)


def _combine_body(p_ref, scale_ref, o_ref):
    o_ref[...] = (p_ref[0] + p_ref[1]) * scale_ref[...][:, None]


def _tc_combine(p, scale, npad):
    g = npad // LB
    return pl.pallas_call(
        _combine_body,
        grid=(g,),
        in_specs=[
            pl.BlockSpec((NC, LB, LB), lambda i: (0, i, 0)),
            pl.BlockSpec((LB,), lambda i: (i,)),
        ],
        out_specs=pl.BlockSpec((LB, LB), lambda i: (i, 0)),
        out_shape=jax.ShapeDtypeStruct((npad, LB), jnp.float32),
    )(p, scale)


def _layer1_body(sx0_ref, sx1_ref, sx2_ref, th_ref, isdv_ref,
                 y0_ref, y1_ref, y2_ref, y3_ref):
    h = jnp.dot(sx0_ref[...], th_ref[0:LB, :],
                preferred_element_type=jnp.float32)
    h += jnp.dot(sx1_ref[...], th_ref[LB:2 * LB, :],
                 preferred_element_type=jnp.float32)
    h += jnp.dot(sx2_ref[...], th_ref[2 * LB:3 * LB, :],
                 preferred_element_type=jnp.float32)
    y = jnp.maximum(h, 0.0) * isdv_ref[...][:, None]
    y0_ref[...] = y[:, 0:LB]
    y1_ref[...] = y[:, LB:2 * LB]
    y2_ref[...] = y[:, 2 * LB:3 * LB]
    y3_ref[...] = y[:, 3 * LB:4 * LB]


def _tc_layer1(sx0, sx1, sx2, th1a, isdv, npad, dh):
    g = npad // LB
    blk = pl.BlockSpec((LB, LB), lambda i: (i, 0))
    out = jax.ShapeDtypeStruct((npad, LB), jnp.float32)
    return pl.pallas_call(
        _layer1_body,
        grid=(g,),
        in_specs=[
            blk, blk, blk,
            pl.BlockSpec((3 * LB, dh), lambda i: (0, 0)),
            pl.BlockSpec((LB,), lambda i: (i,)),
        ],
        out_specs=(blk, blk, blk, blk),
        out_shape=(out, out, out, out),
    )(sx0, sx1, sx2, th1a, isdv)


def _layer2_body(sh0_ref, sh1_ref, sh2_ref, sh3_ref, th_ref, s_ref, b2_ref,
                 wh_ref, bh_ref, o_ref):
    acc = jnp.dot(sh0_ref[...], th_ref[0:LB, :],
                  preferred_element_type=jnp.float32)
    acc += jnp.dot(sh1_ref[...], th_ref[LB:2 * LB, :],
                   preferred_element_type=jnp.float32)
    acc += jnp.dot(sh2_ref[...], th_ref[2 * LB:3 * LB, :],
                   preferred_element_type=jnp.float32)
    acc += jnp.dot(sh3_ref[...], th_ref[3 * LB:4 * LB, :],
                   preferred_element_type=jnp.float32)
    h2 = jnp.maximum(acc + s_ref[...][:, None] * b2_ref[...][None, :], 0.0)
    o_ref[...] = jnp.dot(h2, wh_ref[...],
                         preferred_element_type=jnp.float32) + bh_ref[0]


def _tc_layer2(sh, th2, s, b2, wh, bh, npad, dh):
    g = npad // LB
    blk = pl.BlockSpec((LB, LB), lambda i: (i, 0))
    return pl.pallas_call(
        _layer2_body,
        grid=(g,),
        in_specs=[
            blk, blk, blk, blk,
            pl.BlockSpec((dh, dh), lambda i: (0, 0)),
            pl.BlockSpec((LB,), lambda i: (i,)),
            pl.BlockSpec((dh,), lambda i: (0,)),
            pl.BlockSpec((dh, 1), lambda i: (0, 0)),
            pl.BlockSpec((1,), lambda i: (0,)),
        ],
        out_specs=pl.BlockSpec((LB, 1), lambda i: (i, 0)),
        out_shape=jax.ShapeDtypeStruct((npad, 1), jnp.float32),
    )(*sh, th2, s, b2, wh, bh)


def kernel(X, hyperedge_index, Theta1, b1, Theta2, b2, Wh, bh):
    n, d_in = X.shape
    dh = Theta1.shape[1]
    e = hyperedge_index.shape[1]
    npad = -(-(n + 1) // LB) * LB
    nb = -(-e // (NW * LB))
    epad = nb * NW * LB

    f32 = jnp.float32
    pad = jnp.full((epad - e,), n, jnp.int32)
    vp = jnp.concatenate([hyperedge_index[0], pad]).reshape(NW, nb, LB)
    ep = jnp.concatenate([hyperedge_index[1], pad]).reshape(NW, nb, LB)

    xp = jnp.zeros((npad, 3 * LB), f32).at[:n, :d_in].set(X)
    th1a = jnp.zeros((3 * LB, dh), f32).at[:d_in].set(Theta1).at[d_in].set(b1)
    zero_lb = jnp.zeros((LB, LB), f32)
    ones_lb = jnp.ones((LB, LB), f32)

    count = _sc_count(nb, npad)
    dvp = count(vp, ones_lb, zero_lb)
    dep = count(ep, ones_lb, zero_lb)
    y1, isdv, ide = _tc_scales_y1(xp, dvp, dep, npad)

    push = _sc_push(nb, npad)

    def smooth_blocks(blocks):
        out = []
        for blk in blocks:
            epart = push(blk, vp, ep, zero_lb)
            edge = _tc_combine(epart, ide, npad)
            npart = push(edge, ep, vp, zero_lb)
            out.append(_tc_combine(npart, isdv, npad))
        return out

    sx = smooth_blocks([y1[k] for k in range(3)])
    s = sx[2][:, 0]
    y2 = _tc_layer1(sx[0], sx[1], sx[2], th1a, isdv, npad, dh)
    sh = smooth_blocks(list(y2))
    out = _tc_layer2(sh, Theta2, s, b2, Wh, bh, npad, dh)
    return out[:n]


# speedup vs baseline: 1.7773x; 1.0706x over previous
import functools

import jax
import jax.numpy as jnp
from jax import lax
from jax.experimental import pallas as pl
from jax.experimental.pallas import tpu as pltpu
from jax.experimental.pallas import tpu_sc as plsc

LB = 128
NC = 2
NS = 16
NW = NC * NS


def _chunks(total, step):
    out = []
    t = 0
    while t < total:
        out.append((t, min(step, total - t)))
        t += step
    return out


@functools.lru_cache(maxsize=None)
def _sc_push(nb, npad):
    rpt = npad // NS
    mesh = plsc.VectorSubcoreMesh(core_axis_name="c", subcore_axis_name="s")

    @functools.partial(
        pl.kernel,
        out_type=jax.ShapeDtypeStruct((NC, npad, LB), jnp.float32),
        mesh=mesh,
        scratch_types=[
            pltpu.VMEM((nb, LB), jnp.int32),
            pltpu.VMEM((nb, LB), jnp.int32),
            pltpu.VMEM((2, LB, LB), jnp.float32),
            pltpu.VMEM_SHARED((npad, LB), jnp.float32),
            pltpu.SemaphoreType.DMA((2,)),
            pltpu.SemaphoreType.DMA((2,)),
        ],
    )
    def push(y_hbm, sidx_hbm, didx_hbm, zero_hbm, out_hbm,
             sidx, didx, rows, acc, gsem, ssem):
        c = lax.axis_index("c")
        s = lax.axis_index("s")
        wid = c * NS + s
        pltpu.sync_copy(sidx_hbm.at[wid], sidx)
        pltpu.sync_copy(didx_hbm.at[wid], didx)
        pltpu.sync_copy(zero_hbm, rows.at[0])
        base = s * rpt
        for t0, sz in _chunks(rpt, LB):
            pltpu.sync_copy(rows.at[0, pl.ds(0, sz)],
                            acc.at[pl.ds(base + t0, sz)])
        plsc.subcore_barrier()

        pltpu.async_copy(y_hbm.at[sidx.at[0]], rows.at[0], gsem.at[0])

        @pl.loop(0, nb)
        def _(j):
            b = lax.rem(j, 2)
            o = 1 - b
            pltpu.make_async_copy(y_hbm.at[sidx.at[j]], rows.at[b],
                                  gsem.at[b]).wait()

            @pl.when(j >= 1)
            def _():
                pltpu.make_async_copy(rows.at[o], acc.at[didx.at[j - 1]],
                                      ssem.at[o]).wait()

            @pl.when(j + 1 < nb)
            def _():
                pltpu.async_copy(y_hbm.at[sidx.at[j + 1]], rows.at[o],
                                 gsem.at[o])

            pltpu.async_copy(rows.at[b], acc.at[didx.at[j]], ssem.at[b],
                             add=True)

        last = (nb - 1) % 2
        pltpu.make_async_copy(rows.at[last], acc.at[didx.at[nb - 1]],
                              ssem.at[last]).wait()
        plsc.subcore_barrier()
        for t0, sz in _chunks(rpt, LB):
            pltpu.sync_copy(acc.at[pl.ds(base + t0, sz)],
                            rows.at[0, pl.ds(0, sz)])
            pltpu.sync_copy(rows.at[0, pl.ds(0, sz)],
                            out_hbm.at[c, pl.ds(base + t0, sz)])

    return push


@functools.lru_cache(maxsize=None)
def _sc_count(nb, npad):
    rpt = npad // NS
    mesh = plsc.VectorSubcoreMesh(core_axis_name="c", subcore_axis_name="s")

    @functools.partial(
        pl.kernel,
        out_type=jax.ShapeDtypeStruct((NC, npad, LB), jnp.float32),
        mesh=mesh,
        scratch_types=[
            pltpu.VMEM((nb, LB), jnp.int32),
            pltpu.VMEM((LB, LB), jnp.float32),
            pltpu.VMEM_SHARED((npad, LB), jnp.float32),
            pltpu.SemaphoreType.DMA,
        ],
    )
    def count(idx_hbm, ones_hbm, zero_hbm, out_hbm, idx, buf, acc, sem):
        c = lax.axis_index("c")
        s = lax.axis_index("s")
        wid = c * NS + s
        pltpu.sync_copy(idx_hbm.at[wid], idx)
        pltpu.sync_copy(zero_hbm, buf)
        base = s * rpt
        for t0, sz in _chunks(rpt, LB):
            pltpu.sync_copy(buf.at[pl.ds(0, sz)], acc.at[pl.ds(base + t0, sz)])
        plsc.subcore_barrier()
        pltpu.sync_copy(ones_hbm, buf)

        @pl.loop(0, nb)
        def _(j):
            pltpu.async_copy(buf, acc.at[idx.at[j]], sem, add=True)

        @pl.loop(0, nb)
        def _(j):
            pltpu.make_async_copy(buf, acc.at[idx.at[0]], sem).wait()

        plsc.subcore_barrier()
        for t0, sz in _chunks(rpt, LB):
            pltpu.sync_copy(acc.at[pl.ds(base + t0, sz)], buf.at[pl.ds(0, sz)])
            pltpu.sync_copy(buf.at[pl.ds(0, sz)],
                            out_hbm.at[c, pl.ds(base + t0, sz)])

    return count


def _scales_y1_body(x_ref, dvp_ref, dep_ref, y1_ref, isdv_ref, ide_ref):
    k = pl.program_id(0)
    dv = dvp_ref[0, :, 0] + dvp_ref[1, :, 0]
    de = dep_ref[0, :, 0] + dep_ref[1, :, 0]
    isdv = jnp.where(dv > 0, lax.rsqrt(jnp.maximum(dv, 1.0)), 0.0)
    ide = jnp.where(de > 0, 1.0 / jnp.maximum(de, 1.0), 0.0)
    isdv_ref[...] = isdv
    ide_ref[...] = ide
    onehot = (lax.broadcasted_iota(jnp.int32, (1, LB), 1) == 0)
    aug = jnp.where(k == 2, onehot.astype(jnp.float32), 0.0)
    y1_ref[...] = ((x_ref[...] + aug) * isdv[:, None])[None]


def _tc_scales_y1(xp, dvp, dep, npad):
    g = npad // LB
    return pl.pallas_call(
        _scales_y1_body,
        grid=(3, g),
        in_specs=[
            pl.BlockSpec((LB, LB), lambda k, i: (i, k)),
            pl.BlockSpec((NC, LB, LB), lambda k, i: (0, i, 0)),
            pl.BlockSpec((NC, LB, LB), lambda k, i: (0, i, 0)),
        ],
        out_specs=(
            pl.BlockSpec((1, LB, LB), lambda k, i: (k, i, 0)),
            pl.BlockSpec((LB,), lambda k, i: (i,)),
            pl.BlockSpec((LB,), lambda k, i: (i,)),
        ),
        out_shape=(
            jax.ShapeDtypeStruct((3, npad, LB), jnp.float32),
            jax.ShapeDtypeStruct((npad,), jnp.float32),
            jax.ShapeDtypeStruct((npad,), jnp.float32),
        ),
    )(xp, dvp, dep)


def _combine_body(p_ref, scale_ref, o_ref):
    o_ref[...] = (p_ref[0] + p_ref[1]) * scale_ref[...][:, None]


def _tc_combine(p, scale, npad):
    g = npad // LB
    return pl.pallas_call(
        _combine_body,
        grid=(g,),
        in_specs=[
            pl.BlockSpec((NC, LB, LB), lambda i: (0, i, 0)),
            pl.BlockSpec((LB,), lambda i: (i,)),
        ],
        out_specs=pl.BlockSpec((LB, LB), lambda i: (i, 0)),
        out_shape=jax.ShapeDtypeStruct((npad, LB), jnp.float32),
    )(p, scale)


def _layer1_body(sx0_ref, sx1_ref, sx2_ref, th_ref, isdv_ref,
                 y0_ref, y1_ref, y2_ref, y3_ref):
    h = jnp.dot(sx0_ref[...], th_ref[0:LB, :],
                preferred_element_type=jnp.float32)
    h += jnp.dot(sx1_ref[...], th_ref[LB:2 * LB, :],
                 preferred_element_type=jnp.float32)
    h += jnp.dot(sx2_ref[...], th_ref[2 * LB:3 * LB, :],
                 preferred_element_type=jnp.float32)
    y = jnp.maximum(h, 0.0) * isdv_ref[...][:, None]
    y0_ref[...] = y[:, 0:LB]
    y1_ref[...] = y[:, LB:2 * LB]
    y2_ref[...] = y[:, 2 * LB:3 * LB]
    y3_ref[...] = y[:, 3 * LB:4 * LB]


def _tc_layer1(sx0, sx1, sx2, th1a, isdv, npad, dh):
    g = npad // LB
    blk = pl.BlockSpec((LB, LB), lambda i: (i, 0))
    out = jax.ShapeDtypeStruct((npad, LB), jnp.float32)
    return pl.pallas_call(
        _layer1_body,
        grid=(g,),
        in_specs=[
            blk, blk, blk,
            pl.BlockSpec((3 * LB, dh), lambda i: (0, 0)),
            pl.BlockSpec((LB,), lambda i: (i,)),
        ],
        out_specs=(blk, blk, blk, blk),
        out_shape=(out, out, out, out),
    )(sx0, sx1, sx2, th1a, isdv)


def _layer2_body(sh0_ref, sh1_ref, sh2_ref, sh3_ref, th_ref, s_ref, b2_ref,
                 wh_ref, bh_ref, o_ref):
    acc = jnp.dot(sh0_ref[...], th_ref[0:LB, :],
                  preferred_element_type=jnp.float32)
    acc += jnp.dot(sh1_ref[...], th_ref[LB:2 * LB, :],
                   preferred_element_type=jnp.float32)
    acc += jnp.dot(sh2_ref[...], th_ref[2 * LB:3 * LB, :],
                   preferred_element_type=jnp.float32)
    acc += jnp.dot(sh3_ref[...], th_ref[3 * LB:4 * LB, :],
                   preferred_element_type=jnp.float32)
    h2 = jnp.maximum(acc + s_ref[...][:, None] * b2_ref[...][None, :], 0.0)
    o_ref[...] = jnp.dot(h2, wh_ref[...],
                         preferred_element_type=jnp.float32) + bh_ref[0]


def _tc_layer2(sh, th2, s, b2, wh, bh, npad, dh):
    g = npad // LB
    blk = pl.BlockSpec((LB, LB), lambda i: (i, 0))
    return pl.pallas_call(
        _layer2_body,
        grid=(g,),
        in_specs=[
            blk, blk, blk, blk,
            pl.BlockSpec((dh, dh), lambda i: (0, 0)),
            pl.BlockSpec((LB,), lambda i: (i,)),
            pl.BlockSpec((dh,), lambda i: (0,)),
            pl.BlockSpec((dh, 1), lambda i: (0, 0)),
            pl.BlockSpec((1,), lambda i: (0,)),
        ],
        out_specs=pl.BlockSpec((LB, 1), lambda i: (i, 0)),
        out_shape=jax.ShapeDtypeStruct((npad, 1), jnp.float32),
    )(*sh, th2, s, b2, wh, bh)


def kernel(X, hyperedge_index, Theta1, b1, Theta2, b2, Wh, bh):
    n, d_in = X.shape
    dh = Theta1.shape[1]
    e = hyperedge_index.shape[1]
    npad = -(-(n + 1) // LB) * LB
    nb = -(-e // (NW * LB))
    epad = nb * NW * LB

    f32 = jnp.float32
    pad = jnp.full((epad - e,), n, jnp.int32)
    vp = jnp.concatenate([hyperedge_index[0], pad]).reshape(NW, nb, LB)
    ep = jnp.concatenate([hyperedge_index[1], pad]).reshape(NW, nb, LB)

    xp = jnp.zeros((npad, 3 * LB), f32).at[:n, :d_in].set(X)
    th1a = jnp.zeros((3 * LB, dh), f32).at[:d_in].set(Theta1).at[d_in].set(b1)
    zero_lb = jnp.zeros((LB, LB), f32)
    ones_lb = jnp.ones((LB, LB), f32)

    count = _sc_count(nb, npad)
    dvp = count(vp, ones_lb, zero_lb)
    dep = count(ep, ones_lb, zero_lb)
    y1, isdv, ide = _tc_scales_y1(xp, dvp, dep, npad)

    push = _sc_push(nb, npad)

    def smooth_blocks(blocks):
        out = []
        for blk in blocks:
            epart = push(blk, vp, ep, zero_lb)
            edge = _tc_combine(epart, ide, npad)
            npart = push(edge, ep, vp, zero_lb)
            out.append(_tc_combine(npart, isdv, npad))
        return out

    sx = smooth_blocks([y1[k] for k in range(3)])
    s = sx[2][:, 0]
    y2 = _tc_layer1(sx[0], sx[1], sx[2], th1a, isdv, npad, dh)
    sh = smooth_blocks(list(y2))
    out = _tc_layer2(sh, Theta2, s, b2, Wh, bh, npad, dh)
    return out[:n]
